# Initial kernel scaffold; baseline (speedup 1.0000x reference)
import jax
import jax.numpy as jnp
from jax.experimental import pallas as pl


def kernel(nodes_atoms, nodes_bonds, nodes_monosacchs, batch_ids_atoms, batch_ids_bonds, batch_ids_monosacchs, gate_W, gate_b, W1, b1, prelu_a, bn_gamma, bn_beta, W2, b2):
    raise NotImplementedError("write your pallas kernel here")


# speedup vs baseline: 5.3860x; 5.3860x over previous
import jax
import jax.numpy as jnp
from jax.experimental import pallas as pl
from jax.experimental.pallas import tpu as pltpu

S = 256


def _row_block(n: int, cap: int = 2048) -> int:
    r = 8
    for c in range(8, cap + 1, 8):
        if n % c == 0:
            r = c
    return r


def _gate_stats_body(x_ref, gw_ref, gb_ref, ids_ref, gate_ref, m_ref, s_ref):
    i = pl.program_id(0)
    g = jax.lax.dot_general(x_ref[...], gw_ref[...], (((1,), (0,)), ((), ())),
                            preferred_element_type=jnp.float32) + gb_ref[...]
    gate_ref[...] = g
    bm = jnp.max(g).reshape(1, 1)
    prev_m = jnp.where(i == 0, jnp.full((1, 1), -1e30, jnp.float32), m_ref[...])
    prev_s = jnp.where(i == 0, jnp.zeros_like(s_ref[...]), s_ref[...])
    m1 = jnp.maximum(prev_m, bm)
    e = jnp.exp(g - m1)
    oh = (ids_ref[...] == jax.lax.broadcasted_iota(jnp.int32, (1, S), 1))
    contrib = jnp.sum(oh.astype(jnp.float32) * e, axis=0, keepdims=True)
    s_ref[...] = prev_s * jnp.exp(prev_m - m1) + contrib
    m_ref[...] = m1


def _main_body(x_ref, w1_ref, b1_ref, a_ref, ids_ref, gate_ref, m_ref, s_ref,
               acc_ref):
    i = pl.program_id(0)
    e = jnp.exp(gate_ref[...] - m_ref[...])
    oh = (ids_ref[...] == jax.lax.broadcasted_iota(jnp.int32, (1, S), 1))
    ohf = oh.astype(jnp.float32)
    sg = jnp.sum(ohf * s_ref[...], axis=1, keepdims=True)
    w = e / (sg + 1e-16)
    u = jax.lax.dot_general(x_ref[...], w1_ref[...], (((1,), (0,)), ((), ())),
                            preferred_element_type=jnp.float32) + b1_ref[...]
    u = jnp.where(u >= 0, u, a_ref[...] * u)
    contrib = jax.lax.dot_general(ohf * w, u, (((0,), (0,)), ((), ())),
                                  preferred_element_type=jnp.float32)

    @pl.when(i == 0)
    def _():
        acc_ref[...] = contrib

    @pl.when(i > 0)
    def _():
        acc_ref[...] += contrib


def _epilogue_body(aa_ref, ab_ref, am_ref, sc_ref, gsc_ref, beta_ref, w2_ref,
                   b2_ref, out_ref):
    acc = aa_ref[...] + ab_ref[...] + am_ref[...]
    sw = sc_ref[...] / (sc_ref[...] + 1e-16)
    z = acc * gsc_ref[...] + sw * beta_ref[...]
    out_ref[...] = jax.lax.dot_general(
        z, w2_ref[...], (((1,), (0,)), ((), ())),
        preferred_element_type=jnp.float32) + sw * b2_ref[...]


def _gate_stats(x, ids_col, gw_col, gb11):
    n, d = x.shape
    r = _row_block(n)
    nb = n // r
    return pl.pallas_call(
        _gate_stats_body,
        grid=(nb,),
        in_specs=[
            pl.BlockSpec((r, d), lambda i: (i, 0)),
            pl.BlockSpec((d, 1), lambda i: (0, 0)),
            pl.BlockSpec((1, 1), lambda i: (0, 0)),
            pl.BlockSpec((r, 1), lambda i: (i, 0)),
        ],
        out_specs=[
            pl.BlockSpec((r, 1), lambda i: (i, 0)),
            pl.BlockSpec((1, 1), lambda i: (0, 0)),
            pl.BlockSpec((1, S), lambda i: (0, 0)),
        ],
        out_shape=[
            jax.ShapeDtypeStruct((n, 1), jnp.float32),
            jax.ShapeDtypeStruct((1, 1), jnp.float32),
            jax.ShapeDtypeStruct((1, S), jnp.float32),
        ],
        compiler_params=pltpu.CompilerParams(
            dimension_semantics=("arbitrary",)),
    )(x, gw_col, gb11, ids_col)


def _main(x, ids_col, gate_col, w1, b1_row, a11, m11, s_row):
    n, d = x.shape
    r = _row_block(n)
    nb = n // r
    return pl.pallas_call(
        _main_body,
        grid=(nb,),
        in_specs=[
            pl.BlockSpec((r, d), lambda i: (i, 0)),
            pl.BlockSpec((d, d), lambda i: (0, 0)),
            pl.BlockSpec((1, d), lambda i: (0, 0)),
            pl.BlockSpec((1, 1), lambda i: (0, 0)),
            pl.BlockSpec((r, 1), lambda i: (i, 0)),
            pl.BlockSpec((r, 1), lambda i: (i, 0)),
            pl.BlockSpec((1, 1), lambda i: (0, 0)),
            pl.BlockSpec((1, S), lambda i: (0, 0)),
        ],
        out_specs=pl.BlockSpec((S, d), lambda i: (0, 0)),
        out_shape=jax.ShapeDtypeStruct((S, d), jnp.float32),
        compiler_params=pltpu.CompilerParams(
            dimension_semantics=("arbitrary",)),
    )(x, w1, b1_row, a11, ids_col, gate_col, m11, s_row)


def kernel(nodes_atoms, nodes_bonds, nodes_monosacchs, batch_ids_atoms,
           batch_ids_bonds, batch_ids_monosacchs, gate_W, gate_b, W1, b1,
           prelu_a, bn_gamma, bn_beta, W2, b2):
    d = nodes_atoms.shape[1]
    parts = [
        (nodes_atoms, batch_ids_atoms),
        (nodes_bonds, batch_ids_bonds),
        (nodes_monosacchs, batch_ids_monosacchs),
    ]
    gw_col = gate_W.reshape(d, 1).astype(jnp.float32)
    gb11 = gate_b.reshape(1, 1).astype(jnp.float32)
    a11 = jnp.asarray(prelu_a, jnp.float32).reshape(1, 1)
    b1_row = b1.reshape(1, d)
    b2_row = b2.reshape(1, d)
    beta_row = bn_beta.reshape(1, d)
    gscale_row = (bn_gamma * (1.0 / jnp.sqrt(1.0 + 1e-5))).reshape(1, d)

    ids_cols, gates, ms, ss = [], [], [], []
    for x, ids in parts:
        ids_col = ids.astype(jnp.int32).reshape(-1, 1)
        gate_col, m11, s_row = _gate_stats(x, ids_col, gw_col, gb11)
        ids_cols.append(ids_col)
        gates.append(gate_col)
        ms.append(m11)
        ss.append(s_row)

    m_glob = jnp.maximum(jnp.maximum(ms[0], ms[1]), ms[2])
    s_tot = sum(s * jnp.exp(m - m_glob) for s, m in zip(ss, ms))

    accs = [
        _main(x, ids_cols[k], gates[k], W1, b1_row, a11, m_glob, s_tot)
        for k, (x, _) in enumerate(parts)
    ]

    s_col = s_tot.reshape(S, 1)
    out = pl.pallas_call(
        _epilogue_body,
        out_shape=jax.ShapeDtypeStruct((S, d), jnp.float32),
    )(accs[0], accs[1], accs[2], s_col, gscale_row, beta_row, W2, b2_row)
    return out
